# Initial kernel scaffold; baseline (speedup 1.0000x reference)
import math

import jax
import jax.numpy as jnp
from jax.experimental import pallas as pl
from jax.experimental.pallas import tpu as pltpu

N = 64
IN_DIM = 128
P = 256
H = 128
HEADS = 4
DH = H // HEADS
K = 32
LAYERS = 2

_INTERPRET = False


def _dotf(a, b):
    return jnp.dot(a, b, preferred_element_type=jnp.float32)


def _dot_nt(a, b):
    return jax.lax.dot_general(a, b, (((1,), (1,)), ((), ())),
                               preferred_element_type=jnp.float32)


def _pairwise_d2(posc, posr):
    d2 = jnp.zeros((P, P), jnp.float32)
    for c in range(3):
        diff = posc[:, c:c + 1] - posr[c:c + 1, :]
        d2 = d2 + diff * diff
    ri = jax.lax.broadcasted_iota(jnp.int32, (P, P), 0)
    ci = jax.lax.broadcasted_iota(jnp.int32, (P, P), 1)
    return jnp.where(ri == ci, 1e10, d2)


def _knn_mask(d2):
    work = d2
    mask = jnp.zeros((P, P), jnp.float32)
    for _ in range(K):
        m = jnp.min(work, axis=1, keepdims=True)
        sel = work <= m
        mask = jnp.where(sel, 1.0, mask)
        work = jnp.where(sel, jnp.float32(3e38), work)
    return mask


def _embed_kernel(x_ref, w_ref, b_ref, o_ref):
    o_ref[...] = _dotf(x_ref[...], w_ref[...]) + b_ref[...]


def _mask0_kernel(posc_ref, posr_ref, mask_ref):
    d2 = _pairwise_d2(posc_ref[...], posr_ref[...])
    mask_ref[...] = _knn_mask(d2)


def _graph_kernel(h0_ref, posc_ref, posr_ref, eaT_ref, mask0_ref,
                  wq_ref, bq_ref, wk_ref, bk_ref, wv_ref, bv_ref,
                  wo_ref, bo_ref, lin_w_ref, lin_b_ref,
                  w1a_ref, w1b_ref, w_ea_ref, w_d_ref, e_b1_ref,
                  e_w2_ref, e_b2_ref,
                  n_w1h_ref, n_w1e_ref, n_b1_ref, n_w2_ref, n_b2_ref,
                  c_w1_ref, c_b1_ref, c_w2_ref, c_b2_ref,
                  out1_ref, out2_ref):
    h = h0_ref[0]

    q = _dotf(h, wq_ref[...]) + bq_ref[...]
    k = _dotf(h, wk_ref[...]) + bk_ref[...]
    v = _dotf(h, wv_ref[...]) + bv_ref[...]
    scale = 1.0 / math.sqrt(DH)
    heads = []
    for hd in range(HEADS):
        sl = slice(hd * DH, (hd + 1) * DH)
        s = _dot_nt(q[:, sl], k[:, sl]) * scale
        s = s - jnp.max(s, axis=1, keepdims=True)
        e = jnp.exp(s)
        pattn = e / jnp.sum(e, axis=1, keepdims=True)
        heads.append(_dotf(pattn, v[:, sl]))
    o = jnp.concatenate(heads, axis=1)
    o = _dotf(o, wo_ref[...]) + bo_ref[...]
    h = _dotf(o, lin_w_ref[...]) + lin_b_ref[...]

    eaT = eaT_ref[...]
    posc = posc_ref[...]
    posr = posr_ref[...]
    out_refs = (out1_ref, out2_ref)

    for l in range(LAYERS):
        d2 = _pairwise_d2(posc, posr)
        dist = jnp.sqrt(d2)
        if l == 0:
            mask = mask0_ref[...]
        else:
            mask = _knn_mask(d2)

        a = _dotf(h, w1a_ref[l]) + e_b1_ref[l]
        b = _dotf(h, w1b_ref[l])
        bT = b.T
        w_ea = w_ea_ref[l]
        w_d = w_d_ref[l]
        w2 = e_w2_ref[l]
        F = jnp.zeros((P, P), jnp.float32)
        for m in range(16):
            u = (a[:, m:m + 1] + bT[m:m + 1, :]
                 + eaT * w_ea[0:1, m:m + 1]
                 + dist * w_d[0:1, m:m + 1])
            F = F + jnp.maximum(u, 0.0) * w2[0:1, m:m + 1]
        s = jnp.sum(mask * F, axis=1, keepdims=True)
        e_mean = s * (1.0 / K) + e_b2_ref[l]

        pre = _dotf(h, n_w1h_ref[l]) + e_mean * n_w1e_ref[l] + n_b1_ref[l]
        h = _dotf(jnp.maximum(pre, 0.0), n_w2_ref[l]) + n_b2_ref[l]

        g = jnp.maximum(e_mean * c_w1_ref[l] + c_b1_ref[l], 0.0)
        dpos = _dotf(g, c_w2_ref[l]) + c_b2_ref[l]
        posc = posc + dpos
        posr = posc.T
        out_refs[l][0] = posc


def _full(shape):
    rank = len(shape)
    return pl.BlockSpec(shape, lambda n: (0,) * rank)


@jax.jit
def kernel(x, pos, edge_attr, params):
    f32 = jnp.float32

    CB = 4096
    n_cb = (P * H) // CB
    h0 = pl.pallas_call(
        _embed_kernel,
        grid=(n_cb,),
        in_specs=[
            pl.BlockSpec((N, IN_DIM), lambda i: (0, 0)),
            pl.BlockSpec((IN_DIM, CB), lambda i: (0, i)),
            pl.BlockSpec((1, CB), lambda i: (0, i)),
        ],
        out_specs=pl.BlockSpec((N, CB), lambda i: (0, i)),
        out_shape=jax.ShapeDtypeStruct((N, P * H), f32),
        compiler_params=pltpu.CompilerParams(
            dimension_semantics=("arbitrary",)),
        interpret=_INTERPRET,
    )(x, params['z_W'], params['z_b'].reshape(1, P * H))
    h0 = h0.reshape(N, P, H)

    posc = pos.astype(f32)
    posr = posc.T

    mask0 = pl.pallas_call(
        _mask0_kernel,
        in_specs=[_full((P, 3)), _full((3, P))],
        out_specs=_full((P, P)),
        out_shape=jax.ShapeDtypeStruct((P, P), f32),
        interpret=_INTERPRET,
    )(posc, posr)

    lp = params['layers']

    def stack(name):
        return jnp.stack([lp[l][name] for l in range(LAYERS)])

    e_W1 = stack('e_W1')
    w1a = e_W1[:, :H, :]
    w1b = e_W1[:, H:2 * H, :]
    w_ea = e_W1[:, 2 * H:2 * H + 1, :]
    w_d = e_W1[:, 2 * H + 1:2 * H + 2, :]
    e_b1 = stack('e_b1').reshape(LAYERS, 1, 16)
    e_w2 = stack('e_W2').reshape(LAYERS, 1, 16)
    e_b2 = stack('e_b2').reshape(LAYERS, 1, 1)
    n_W1 = stack('n_W1')
    n_w1h = n_W1[:, :H, :]
    n_w1e = n_W1[:, H:H + 1, :]
    n_b1 = stack('n_b1').reshape(LAYERS, 1, 16)
    n_w2 = stack('n_W2')
    n_b2 = stack('n_b2').reshape(LAYERS, 1, H)
    c_w1 = stack('c_W1')
    c_b1 = stack('c_b1').reshape(LAYERS, 1, 16)
    c_w2 = stack('c_W2')
    c_b2 = stack('c_b2').reshape(LAYERS, 1, 3)

    in_specs = [
        pl.BlockSpec((1, P, H), lambda n: (n, 0, 0)),
        _full((P, 3)), _full((3, P)), _full((P, P)), _full((P, P)),
        _full((H, H)), _full((1, H)), _full((H, H)), _full((1, H)),
        _full((H, H)), _full((1, H)), _full((H, H)), _full((1, H)),
        _full((H, H)), _full((1, H)),
        _full((LAYERS, H, 16)), _full((LAYERS, H, 16)),
        _full((LAYERS, 1, 16)), _full((LAYERS, 1, 16)),
        _full((LAYERS, 1, 16)), _full((LAYERS, 1, 16)),
        _full((LAYERS, 1, 1)),
        _full((LAYERS, H, 16)), _full((LAYERS, 1, 16)),
        _full((LAYERS, 1, 16)), _full((LAYERS, 16, H)),
        _full((LAYERS, 1, H)),
        _full((LAYERS, 1, 16)), _full((LAYERS, 1, 16)),
        _full((LAYERS, 16, 3)), _full((LAYERS, 1, 3)),
    ]
    out_spec = pl.BlockSpec((1, P, 3), lambda n: (n, 0, 0))
    c1, c2 = pl.pallas_call(
        _graph_kernel,
        grid=(N,),
        in_specs=in_specs,
        out_specs=[out_spec, out_spec],
        out_shape=[jax.ShapeDtypeStruct((N, P, 3), f32),
                   jax.ShapeDtypeStruct((N, P, 3), f32)],
        compiler_params=pltpu.CompilerParams(
            dimension_semantics=("parallel",)),
        interpret=_INTERPRET,
    )(h0, posc, posr, edge_attr.T, mask0,
      params['Wq'], params['bq'].reshape(1, H),
      params['Wk'], params['bk'].reshape(1, H),
      params['Wv'], params['bv'].reshape(1, H),
      params['Wo'], params['bo'].reshape(1, H),
      params['lin_W'], params['lin_b'].reshape(1, H),
      w1a, w1b, w_ea, w_d, e_b1, e_w2, e_b2,
      n_w1h, n_w1e, n_b1, n_w2, n_b2,
      c_w1, c_b1, c_w2, c_b2)

    return (c1.reshape(N * P, 3), c2.reshape(N * P, 3))


# speedup vs baseline: 52.4447x; 52.4447x over previous
import math

import jax
import jax.numpy as jnp
from jax.experimental import pallas as pl
from jax.experimental.pallas import tpu as pltpu

N = 64
IN_DIM = 128
P = 256
H = 128
HEADS = 4
DH = H // HEADS
K = 32
LAYERS = 2

_INTERPRET = False


def _dotf(a, b):
    return jnp.dot(a, b, preferred_element_type=jnp.float32)


def _dot_nt(a, b):
    return jax.lax.dot_general(a, b, (((1,), (1,)), ((), ())),
                               preferred_element_type=jnp.float32)


def _pairwise_d2(posc, posr):
    d2 = jnp.zeros((P, P), jnp.float32)
    for c in range(3):
        diff = posc[:, c:c + 1] - posr[c:c + 1, :]
        d2 = d2 + diff * diff
    ri = jax.lax.broadcasted_iota(jnp.int32, (P, P), 0)
    ci = jax.lax.broadcasted_iota(jnp.int32, (P, P), 1)
    return jnp.where(ri == ci, 1e10, d2)


def _knn_mask(d2):
    work = d2
    mask = jnp.zeros((P, P), jnp.float32)
    for _ in range(K):
        m = jnp.min(work, axis=1, keepdims=True)
        sel = work <= m
        mask = jnp.where(sel, 1.0, mask)
        work = jnp.where(sel, jnp.float32(3e38), work)
    return mask


def _embed_kernel(x_ref, w_ref, b_ref, o_ref):
    o_ref[...] = _dotf(x_ref[...], w_ref[...]) + b_ref[...]


def _mask0_kernel(posc_ref, posr_ref, mask_ref):
    d2 = _pairwise_d2(posc_ref[...], posr_ref[...])
    mask_ref[...] = _knn_mask(d2)


def _graph_kernel(h0_ref, posc_ref, posr_ref, eaT_ref, mask0_ref,
                  wq_ref, bq_ref, wk_ref, bk_ref, wv_ref, bv_ref,
                  wo_ref, bo_ref, lin_w_ref, lin_b_ref,
                  w1a_ref, w1b_ref, w_ea_ref, w_d_ref, e_b1_ref,
                  e_w2_ref, e_b2_ref,
                  n_w1h_ref, n_w1e_ref, n_b1_ref, n_w2_ref, n_b2_ref,
                  c_w1_ref, c_b1_ref, c_w2_ref, c_b2_ref,
                  out1_ref, out2_ref):
    h = h0_ref[0]

    q = _dotf(h, wq_ref[...]) + bq_ref[...]
    k = _dotf(h, wk_ref[...]) + bk_ref[...]
    v = _dotf(h, wv_ref[...]) + bv_ref[...]
    scale = 1.0 / math.sqrt(DH)
    heads = []
    for hd in range(HEADS):
        sl = slice(hd * DH, (hd + 1) * DH)
        s = _dot_nt(q[:, sl], k[:, sl]) * scale
        s = s - jnp.max(s, axis=1, keepdims=True)
        e = jnp.exp(s)
        pattn = e / jnp.sum(e, axis=1, keepdims=True)
        heads.append(_dotf(pattn, v[:, sl]))
    o = jnp.concatenate(heads, axis=1)
    o = _dotf(o, wo_ref[...]) + bo_ref[...]
    h = _dotf(o, lin_w_ref[...]) + lin_b_ref[...]

    eaT = eaT_ref[...]
    posc = posc_ref[...]
    posr = posr_ref[...]
    out_refs = (out1_ref, out2_ref)

    for l in range(LAYERS):
        d2 = _pairwise_d2(posc, posr)
        dist = jnp.sqrt(d2)
        if l == 0:
            mask = mask0_ref[...]
        else:
            mask = _knn_mask(d2)

        a = _dotf(h, w1a_ref[l]) + e_b1_ref[l]
        b = _dotf(h, w1b_ref[l])
        bT = b.T
        w_ea = w_ea_ref[l]
        w_d = w_d_ref[l]
        w2 = e_w2_ref[l]
        F = jnp.zeros((P, P), jnp.float32)
        for m in range(16):
            u = (a[:, m:m + 1] + bT[m:m + 1, :]
                 + eaT * w_ea[0:1, m:m + 1]
                 + dist * w_d[0:1, m:m + 1])
            F = F + jnp.maximum(u, 0.0) * w2[0:1, m:m + 1]
        s = jnp.sum(mask * F, axis=1, keepdims=True)
        e_mean = s * (1.0 / K) + e_b2_ref[l]

        pre = _dotf(h, n_w1h_ref[l]) + e_mean * n_w1e_ref[l] + n_b1_ref[l]
        h = _dotf(jnp.maximum(pre, 0.0), n_w2_ref[l]) + n_b2_ref[l]

        g = jnp.maximum(e_mean * c_w1_ref[l] + c_b1_ref[l], 0.0)
        dpos = _dotf(g, c_w2_ref[l]) + c_b2_ref[l]
        posc = posc + dpos
        posr = posc.T
        out_refs[l][0] = posc


def _full(shape):
    rank = len(shape)
    return pl.BlockSpec(shape, lambda *_: (0,) * rank)


@jax.jit
def kernel(x, pos, edge_attr, params):
    f32 = jnp.float32

    CB = 4096
    n_cb = (P * H) // CB
    h0 = pl.pallas_call(
        _embed_kernel,
        grid=(n_cb,),
        in_specs=[
            pl.BlockSpec((N, IN_DIM), lambda i: (0, 0)),
            pl.BlockSpec((IN_DIM, CB), lambda i: (0, i)),
            pl.BlockSpec((1, CB), lambda i: (0, i)),
        ],
        out_specs=pl.BlockSpec((N, CB), lambda i: (0, i)),
        out_shape=jax.ShapeDtypeStruct((N, P * H), f32),
        compiler_params=pltpu.CompilerParams(
            dimension_semantics=("arbitrary",)),
        interpret=_INTERPRET,
    )(x, params['z_W'], params['z_b'].reshape(1, P * H))
    h0 = h0.reshape(N, P, H)

    posc = pos.astype(f32)
    posr = posc.T

    mask0 = pl.pallas_call(
        _mask0_kernel,
        in_specs=[_full((P, 3)), _full((3, P))],
        out_specs=_full((P, P)),
        out_shape=jax.ShapeDtypeStruct((P, P), f32),
        interpret=_INTERPRET,
    )(posc, posr)

    lp = params['layers']

    def stack(name):
        return jnp.stack([lp[l][name] for l in range(LAYERS)])

    e_W1 = stack('e_W1')
    w1a = e_W1[:, :H, :]
    w1b = e_W1[:, H:2 * H, :]
    w_ea = e_W1[:, 2 * H:2 * H + 1, :]
    w_d = e_W1[:, 2 * H + 1:2 * H + 2, :]
    e_b1 = stack('e_b1').reshape(LAYERS, 1, 16)
    e_w2 = stack('e_W2').reshape(LAYERS, 1, 16)
    e_b2 = stack('e_b2').reshape(LAYERS, 1, 1)
    n_W1 = stack('n_W1')
    n_w1h = n_W1[:, :H, :]
    n_w1e = n_W1[:, H:H + 1, :]
    n_b1 = stack('n_b1').reshape(LAYERS, 1, 16)
    n_w2 = stack('n_W2')
    n_b2 = stack('n_b2').reshape(LAYERS, 1, H)
    c_w1 = stack('c_W1')
    c_b1 = stack('c_b1').reshape(LAYERS, 1, 16)
    c_w2 = stack('c_W2')
    c_b2 = stack('c_b2').reshape(LAYERS, 1, 3)

    in_specs = [
        pl.BlockSpec((1, P, H), lambda n: (n, 0, 0)),
        _full((P, 3)), _full((3, P)), _full((P, P)), _full((P, P)),
        _full((H, H)), _full((1, H)), _full((H, H)), _full((1, H)),
        _full((H, H)), _full((1, H)), _full((H, H)), _full((1, H)),
        _full((H, H)), _full((1, H)),
        _full((LAYERS, H, 16)), _full((LAYERS, H, 16)),
        _full((LAYERS, 1, 16)), _full((LAYERS, 1, 16)),
        _full((LAYERS, 1, 16)), _full((LAYERS, 1, 16)),
        _full((LAYERS, 1, 1)),
        _full((LAYERS, H, 16)), _full((LAYERS, 1, 16)),
        _full((LAYERS, 1, 16)), _full((LAYERS, 16, H)),
        _full((LAYERS, 1, H)),
        _full((LAYERS, 1, 16)), _full((LAYERS, 1, 16)),
        _full((LAYERS, 16, 3)), _full((LAYERS, 1, 3)),
    ]
    out_spec = pl.BlockSpec((1, P, 3), lambda n: (n, 0, 0))
    c1, c2 = pl.pallas_call(
        _graph_kernel,
        grid=(N,),
        in_specs=in_specs,
        out_specs=[out_spec, out_spec],
        out_shape=[jax.ShapeDtypeStruct((N, P, 3), f32),
                   jax.ShapeDtypeStruct((N, P, 3), f32)],
        compiler_params=pltpu.CompilerParams(
            dimension_semantics=("parallel",)),
        interpret=_INTERPRET,
    )(h0, posc, posr, edge_attr.T, mask0,
      params['Wq'], params['bq'].reshape(1, H),
      params['Wk'], params['bk'].reshape(1, H),
      params['Wv'], params['bv'].reshape(1, H),
      params['Wo'], params['bo'].reshape(1, H),
      params['lin_W'], params['lin_b'].reshape(1, H),
      w1a, w1b, w_ea, w_d, e_b1, e_w2, e_b2,
      n_w1h, n_w1e, n_b1, n_w2, n_b2,
      c_w1, c_b1, c_w2, c_b2)

    return (c1.reshape(N * P, 3), c2.reshape(N * P, 3))
